# speedup vs baseline: 3.5621x; 3.5621x over previous
import functools

import jax
import jax.numpy as jnp
from jax.experimental import pallas as pl
from jax.experimental.pallas import tpu as pltpu

HIDDEN = 256
NUM_EXPERTS = 4
BLOCK_T = 512


def _moe_block_kernel(x_ref, wg_ref, wcat_ref, b_ref, out_ref):
    xb = x_ref[...]
    logits = jnp.dot(xb, wg_ref[...],
                     preferred_element_type=jnp.float32)
    m = jnp.max(logits, axis=-1, keepdims=True)
    ex = jnp.exp(logits - m)
    probs = ex / jnp.sum(ex, axis=-1, keepdims=True)
    idx = jnp.argmax(logits, axis=-1)
    gate = jnp.max(probs, axis=-1)

    ys = jnp.dot(xb, wcat_ref[...],
                 preferred_element_type=jnp.float32)
    acc = jnp.zeros((xb.shape[0], HIDDEN), jnp.float32)
    for e in range(NUM_EXPERTS):
        sel = (idx == e)[:, None]
        ye = ys[:, e * HIDDEN:(e + 1) * HIDDEN] + b_ref[e][None, :]
        acc = acc + jnp.where(sel, ye, 0.0)
    out_ref[...] = gate[:, None] * acc


def kernel(x, Wg, W, b):
    orig_shape = x.shape
    x2 = x.reshape(-1, orig_shape[-1])
    T = x2.shape[0]
    wcat = W.transpose(1, 0, 2).reshape(HIDDEN, NUM_EXPERTS * HIDDEN)
    grid = (T // BLOCK_T,)
    out = pl.pallas_call(
        _moe_block_kernel,
        grid=grid,
        in_specs=[
            pl.BlockSpec((BLOCK_T, HIDDEN), lambda i: (i, 0)),
            pl.BlockSpec((HIDDEN, NUM_EXPERTS), lambda i: (0, 0)),
            pl.BlockSpec((HIDDEN, NUM_EXPERTS * HIDDEN), lambda i: (0, 0)),
            pl.BlockSpec((NUM_EXPERTS, HIDDEN), lambda i: (0, 0)),
        ],
        out_specs=pl.BlockSpec((BLOCK_T, HIDDEN), lambda i: (i, 0)),
        out_shape=jax.ShapeDtypeStruct((T, HIDDEN), jnp.float32),
    )(x2, Wg, wcat, b)
    return out.reshape(orig_shape)


# speedup vs baseline: 4.6459x; 1.3042x over previous
import functools

import jax
import jax.numpy as jnp
from jax.experimental import pallas as pl
from jax.experimental.pallas import tpu as pltpu

HIDDEN = 256
NUM_EXPERTS = 4
BLOCK_T = 1024


def _moe_block_kernel(x_ref, wg_ref, wcat_ref, b_ref, out_ref):
    xb = x_ref[...]
    logits = jnp.dot(xb, wg_ref[...],
                     preferred_element_type=jnp.float32)
    m = jnp.max(logits, axis=-1, keepdims=True)
    ex = jnp.exp(logits - m)
    probs = ex / jnp.sum(ex, axis=-1, keepdims=True)
    idx = jnp.argmax(logits, axis=-1)
    gate = jnp.max(probs, axis=-1)

    ys = jnp.dot(xb, wcat_ref[...],
                 preferred_element_type=jnp.float32)
    acc = jnp.zeros((xb.shape[0], HIDDEN), jnp.float32)
    for e in range(NUM_EXPERTS):
        sel = (idx == e)[:, None]
        ye = ys[:, e * HIDDEN:(e + 1) * HIDDEN] + b_ref[e][None, :]
        acc = acc + jnp.where(sel, ye, 0.0)
    out_ref[...] = gate[:, None] * acc


def kernel(x, Wg, W, b):
    orig_shape = x.shape
    x2 = x.reshape(-1, orig_shape[-1])
    T = x2.shape[0]
    wcat = W.transpose(1, 0, 2).reshape(HIDDEN, NUM_EXPERTS * HIDDEN)
    grid = (T // BLOCK_T,)
    out = pl.pallas_call(
        _moe_block_kernel,
        grid=grid,
        in_specs=[
            pl.BlockSpec((BLOCK_T, HIDDEN), lambda i: (i, 0)),
            pl.BlockSpec((HIDDEN, NUM_EXPERTS), lambda i: (0, 0)),
            pl.BlockSpec((HIDDEN, NUM_EXPERTS * HIDDEN), lambda i: (0, 0)),
            pl.BlockSpec((NUM_EXPERTS, HIDDEN), lambda i: (0, 0)),
        ],
        out_specs=pl.BlockSpec((BLOCK_T, HIDDEN), lambda i: (i, 0)),
        out_shape=jax.ShapeDtypeStruct((T, HIDDEN), jnp.float32),
    )(x2, Wg, wcat, b)
    return out.reshape(orig_shape)


# speedup vs baseline: 5.2287x; 1.1254x over previous
import functools

import jax
import jax.numpy as jnp
from jax.experimental import pallas as pl
from jax.experimental.pallas import tpu as pltpu

HIDDEN = 256
NUM_EXPERTS = 4
BLOCK_T = 2048


def _moe_block_kernel(x_ref, wg_ref, wcat_ref, b_ref, out_ref):
    xb = x_ref[...]
    logits = jnp.dot(xb, wg_ref[...],
                     preferred_element_type=jnp.float32)
    m = jnp.max(logits, axis=-1, keepdims=True)
    ex = jnp.exp(logits - m)
    probs = ex / jnp.sum(ex, axis=-1, keepdims=True)
    idx = jnp.argmax(logits, axis=-1)
    gate = jnp.max(probs, axis=-1)

    ys = jnp.dot(xb, wcat_ref[...],
                 preferred_element_type=jnp.float32)
    acc = jnp.zeros((xb.shape[0], HIDDEN), jnp.float32)
    for e in range(NUM_EXPERTS):
        sel = (idx == e)[:, None]
        ye = ys[:, e * HIDDEN:(e + 1) * HIDDEN] + b_ref[e][None, :]
        acc = acc + jnp.where(sel, ye, 0.0)
    out_ref[...] = gate[:, None] * acc


def kernel(x, Wg, W, b):
    orig_shape = x.shape
    x2 = x.reshape(-1, orig_shape[-1])
    T = x2.shape[0]
    wcat = W.transpose(1, 0, 2).reshape(HIDDEN, NUM_EXPERTS * HIDDEN)
    grid = (T // BLOCK_T,)
    out = pl.pallas_call(
        _moe_block_kernel,
        grid=grid,
        in_specs=[
            pl.BlockSpec((BLOCK_T, HIDDEN), lambda i: (i, 0)),
            pl.BlockSpec((HIDDEN, NUM_EXPERTS), lambda i: (0, 0)),
            pl.BlockSpec((HIDDEN, NUM_EXPERTS * HIDDEN), lambda i: (0, 0)),
            pl.BlockSpec((NUM_EXPERTS, HIDDEN), lambda i: (0, 0)),
        ],
        out_specs=pl.BlockSpec((BLOCK_T, HIDDEN), lambda i: (i, 0)),
        out_shape=jax.ShapeDtypeStruct((T, HIDDEN), jnp.float32),
    )(x2, Wg, wcat, b)
    return out.reshape(orig_shape)
